# baseline (device time: 87391 ns/iter reference)
import jax
import jax.numpy as jnp
from jax import lax
from jax.experimental import pallas as pl
from jax.experimental.pallas import tpu as pltpu

N_DEV = 4
M = 1024
N = 1024
CH = M // N_DEV


def _gelu(z):
    return 0.5 * z * (1.0 + jnp.tanh(0.7978845608 * (z + 0.044715 * z * z * z)))


def kernel(A, B):
    def body(a_ref, b_ref, out_ref, acc_ref, comm_ref,
             rs_send_sems, rs_recv_sems, ag_send_sems, ag_recv_sems):
        my = lax.axis_index("i")
        left = (my - 1) % N_DEV
        right = (my + 1) % N_DEV

        barrier_sem = pltpu.get_barrier_semaphore()
        for nbr in (left, right):
            pl.semaphore_signal(
                barrier_sem, inc=1,
                device_id=(nbr,), device_id_type=pl.DeviceIdType.MESH,
            )
        pl.semaphore_wait(barrier_sem, 2)

        for c in range(N_DEV):
            acc_ref[c, :, :] = jnp.dot(
                a_ref[pl.ds(c * CH, CH), :], b_ref[:, :],
                preferred_element_type=jnp.float32,
            )

        for s in range(N_DEV - 1):
            sc = (my - s) % N_DEV
            rc = (my - s - 1) % N_DEV
            rdma = pltpu.make_async_remote_copy(
                src_ref=acc_ref.at[sc],
                dst_ref=comm_ref.at[s],
                send_sem=rs_send_sems.at[s],
                recv_sem=rs_recv_sems.at[s],
                device_id=(right,),
                device_id_type=pl.DeviceIdType.MESH,
            )
            rdma.start()
            rdma.wait()
            acc_ref[rc, :, :] = acc_ref[rc, :, :] + comm_ref[s, :, :]

        own = right
        out_ref[own, :, :] = _gelu(acc_ref[own, :, :])

        for s in range(N_DEV - 1):
            gc = (my + 1 - s) % N_DEV
            rdma = pltpu.make_async_remote_copy(
                src_ref=out_ref.at[gc],
                dst_ref=out_ref.at[gc],
                send_sem=ag_send_sems.at[s],
                recv_sem=ag_recv_sems.at[s],
                device_id=(right,),
                device_id_type=pl.DeviceIdType.MESH,
            )
            rdma.start()
            rdma.wait()

    out = pl.pallas_call(
        body,
        out_shape=jax.ShapeDtypeStruct((N_DEV, CH, N), jnp.float32),
        in_specs=[
            pl.BlockSpec(memory_space=pltpu.VMEM),
            pl.BlockSpec(memory_space=pltpu.VMEM),
        ],
        out_specs=pl.BlockSpec(memory_space=pltpu.VMEM),
        scratch_shapes=[
            pltpu.VMEM((N_DEV, CH, N), jnp.float32),
            pltpu.VMEM((N_DEV - 1, CH, N), jnp.float32),
            pltpu.SemaphoreType.DMA((N_DEV - 1,)),
            pltpu.SemaphoreType.DMA((N_DEV - 1,)),
            pltpu.SemaphoreType.DMA((N_DEV - 1,)),
            pltpu.SemaphoreType.DMA((N_DEV - 1,)),
        ],
        compiler_params=pltpu.CompilerParams(collective_id=0),
    )(A, B)
    return out.reshape(M, N)


# device time: 36931 ns/iter; 2.3663x vs baseline; 2.3663x over previous
import jax
import jax.numpy as jnp
from jax import lax
from jax.experimental import pallas as pl
from jax.experimental.pallas import tpu as pltpu

N_DEV = 4
M = 1024
N = 1024
CH = M // N_DEV


def _gelu(z):
    return 0.5 * z * (1.0 + jnp.tanh(0.7978845608 * (z + 0.044715 * z * z * z)))


def kernel(A, B):
    def body(a_ref, b_ref, out_ref,
             acc_ref, rs_snd_ref, rs_rcv_ref, ag_snd_ref, ag_rcv_ref,
             rs_send_sems, rs_recv_sems, ag_send_sems, ag_recv_sems):
        my = lax.axis_index("i")

        barrier_sem = pltpu.get_barrier_semaphore()
        for r in range(1, N_DEV):
            pl.semaphore_signal(
                barrier_sem, inc=1,
                device_id=((my + r) % N_DEV,),
                device_id_type=pl.DeviceIdType.MESH,
            )
        pl.semaphore_wait(barrier_sem, N_DEV - 1)

        for c in range(N_DEV):
            acc_ref[c, :, :] = jnp.dot(
                a_ref[pl.ds(c * CH, CH), :], b_ref[:, :],
                preferred_element_type=jnp.float32,
            )

        rs_rdmas = []
        for r in range(1, N_DEV):
            t = (my + r) % N_DEV
            rs_snd_ref[r - 1, :, :] = acc_ref[t, :, :].astype(jnp.bfloat16)
            rdma = pltpu.make_async_remote_copy(
                src_ref=rs_snd_ref.at[r - 1],
                dst_ref=rs_rcv_ref.at[r - 1],
                send_sem=rs_send_sems.at[r - 1],
                recv_sem=rs_recv_sems.at[r - 1],
                device_id=(t,),
                device_id_type=pl.DeviceIdType.MESH,
            )
            rdma.start()
            rs_rdmas.append(rdma)
        for rdma in rs_rdmas:
            rdma.wait()

        z = acc_ref[my, :, :]
        for q in range(N_DEV - 1):
            z = z + rs_rcv_ref[q, :, :].astype(jnp.float32)
        g = _gelu(z)
        out_ref[my, :, :] = g
        ag_snd_ref[:, :] = g.astype(jnp.bfloat16)

        ag_rdmas = []
        for r in range(1, N_DEV):
            t = (my + r) % N_DEV
            rdma = pltpu.make_async_remote_copy(
                src_ref=ag_snd_ref,
                dst_ref=ag_rcv_ref.at[r - 1],
                send_sem=ag_send_sems.at[r - 1],
                recv_sem=ag_recv_sems.at[r - 1],
                device_id=(t,),
                device_id_type=pl.DeviceIdType.MESH,
            )
            rdma.start()
            ag_rdmas.append(rdma)
        for rdma in ag_rdmas:
            rdma.wait()

        for q in range(N_DEV - 1):
            cq = (my - q - 1) % N_DEV
            out_ref[cq, :, :] = ag_rcv_ref[q, :, :].astype(jnp.float32)

    out = pl.pallas_call(
        body,
        out_shape=jax.ShapeDtypeStruct((N_DEV, CH, N), jnp.float32),
        in_specs=[
            pl.BlockSpec(memory_space=pltpu.VMEM),
            pl.BlockSpec(memory_space=pltpu.VMEM),
        ],
        out_specs=pl.BlockSpec(memory_space=pltpu.VMEM),
        scratch_shapes=[
            pltpu.VMEM((N_DEV, CH, N), jnp.float32),
            pltpu.VMEM((N_DEV - 1, CH, N), jnp.bfloat16),
            pltpu.VMEM((N_DEV - 1, CH, N), jnp.bfloat16),
            pltpu.VMEM((CH, N), jnp.bfloat16),
            pltpu.VMEM((N_DEV - 1, CH, N), jnp.bfloat16),
            pltpu.SemaphoreType.DMA((N_DEV - 1,)),
            pltpu.SemaphoreType.DMA((N_DEV - 1,)),
            pltpu.SemaphoreType.DMA((N_DEV - 1,)),
            pltpu.SemaphoreType.DMA((N_DEV - 1,)),
        ],
        compiler_params=pltpu.CompilerParams(collective_id=0),
    )(A, B)
    return out.reshape(M, N)


# device time: 36420 ns/iter; 2.3995x vs baseline; 1.0140x over previous
import jax
import jax.numpy as jnp
from jax import lax
from jax.experimental import pallas as pl
from jax.experimental.pallas import tpu as pltpu

N_DEV = 4
M = 1024
N = 1024
CH = M // N_DEV


def _gelu(z):
    return 0.5 * z * (1.0 + jnp.tanh(0.7978845608 * (z + 0.044715 * z * z * z)))


def kernel(A, B):
    def body(a_ref, b_ref, out_ref,
             acc_ref, rs_snd_ref, rs_rcv_ref, ag_snd_ref, ag_rcv_ref,
             rs_send_sems, rs_recv_sems, ag_send_sems, ag_recv_sems):
        my = lax.axis_index("i")

        barrier_sem = pltpu.get_barrier_semaphore()
        for r in range(1, N_DEV):
            pl.semaphore_signal(
                barrier_sem, inc=1,
                device_id=((my + r) % N_DEV,),
                device_id_type=pl.DeviceIdType.MESH,
            )
        pl.semaphore_wait(barrier_sem, N_DEV - 1)

        rs_rdmas = {}
        for r in (2, 1, 3):
            t = (my + r) % N_DEV
            rs_snd_ref[r - 1, :, :] = jnp.dot(
                a_ref[pl.ds(t * CH, CH), :], b_ref[:, :],
                preferred_element_type=jnp.float32,
            ).astype(jnp.bfloat16)
            rdma = pltpu.make_async_remote_copy(
                src_ref=rs_snd_ref.at[r - 1],
                dst_ref=rs_rcv_ref.at[r - 1],
                send_sem=rs_send_sems.at[r - 1],
                recv_sem=rs_recv_sems.at[r - 1],
                device_id=(t,),
                device_id_type=pl.DeviceIdType.MESH,
            )
            rdma.start()
            rs_rdmas[r] = rdma

        z = jnp.dot(
            a_ref[pl.ds(my * CH, CH), :], b_ref[:, :],
            preferred_element_type=jnp.float32,
        )

        for q in (0, 2, 1):
            rs_rdmas[q + 1].wait()
            z = z + rs_rcv_ref[q, :, :].astype(jnp.float32)

        g = _gelu(z)
        ag_snd_ref[:, :] = g.astype(jnp.bfloat16)

        ag_rdmas = {}
        for r in (2, 1, 3):
            t = (my + r) % N_DEV
            rdma = pltpu.make_async_remote_copy(
                src_ref=ag_snd_ref,
                dst_ref=ag_rcv_ref.at[r - 1],
                send_sem=ag_send_sems.at[r - 1],
                recv_sem=ag_recv_sems.at[r - 1],
                device_id=(t,),
                device_id_type=pl.DeviceIdType.MESH,
            )
            rdma.start()
            ag_rdmas[r] = rdma

        out_ref[my, :, :] = g
        for q in (0, 2, 1):
            ag_rdmas[q + 1].wait()
            cq = (my - q - 1) % N_DEV
            out_ref[cq, :, :] = ag_rcv_ref[q, :, :].astype(jnp.float32)

    out = pl.pallas_call(
        body,
        out_shape=jax.ShapeDtypeStruct((N_DEV, CH, N), jnp.float32),
        in_specs=[
            pl.BlockSpec(memory_space=pltpu.VMEM),
            pl.BlockSpec(memory_space=pltpu.VMEM),
        ],
        out_specs=pl.BlockSpec(memory_space=pltpu.VMEM),
        scratch_shapes=[
            pltpu.VMEM((N_DEV, CH, N), jnp.float32),
            pltpu.VMEM((N_DEV - 1, CH, N), jnp.bfloat16),
            pltpu.VMEM((N_DEV - 1, CH, N), jnp.bfloat16),
            pltpu.VMEM((CH, N), jnp.bfloat16),
            pltpu.VMEM((N_DEV - 1, CH, N), jnp.bfloat16),
            pltpu.SemaphoreType.DMA((N_DEV - 1,)),
            pltpu.SemaphoreType.DMA((N_DEV - 1,)),
            pltpu.SemaphoreType.DMA((N_DEV - 1,)),
            pltpu.SemaphoreType.DMA((N_DEV - 1,)),
        ],
        compiler_params=pltpu.CompilerParams(collective_id=0),
    )(A, B)
    return out.reshape(M, N)


# device time: 33246 ns/iter; 2.6286x vs baseline; 1.0955x over previous
import jax
import jax.numpy as jnp
from jax import lax
from jax.experimental import pallas as pl
from jax.experimental.pallas import tpu as pltpu

N_DEV = 4
M = 1024
N = 1024
CH = M // N_DEV
HN = N // 2


def _gelu(z):
    return 0.5 * z * (1.0 + jnp.tanh(0.7978845608 * (z + 0.044715 * z * z * z)))


def kernel(A, B):
    def body(a_ref, b_ref, out_ref,
             rs_snd_ref, rs_rcv_ref, ag_snd_ref, ag_rcv_ref,
             rs_send_sems, rs_recv_sems, ag_send_sems, ag_recv_sems):
        my = lax.axis_index("i")

        barrier_sem = pltpu.get_barrier_semaphore()
        for r in range(1, N_DEV):
            pl.semaphore_signal(
                barrier_sem, inc=1,
                device_id=((my + r) % N_DEV,),
                device_id_type=pl.DeviceIdType.MESH,
            )
        pl.semaphore_wait(barrier_sem, N_DEV - 1)

        def rs_rdma(r, h):
            t = (my + r) % N_DEV
            return pltpu.make_async_remote_copy(
                src_ref=rs_snd_ref.at[r - 1, :, pl.ds(h * HN, HN)],
                dst_ref=rs_rcv_ref.at[r - 1, :, pl.ds(h * HN, HN)],
                send_sem=rs_send_sems.at[r - 1, h],
                recv_sem=rs_recv_sems.at[r - 1, h],
                device_id=(t,),
                device_id_type=pl.DeviceIdType.MESH,
            )

        def ag_rdma(r, h):
            t = (my + r) % N_DEV
            return pltpu.make_async_remote_copy(
                src_ref=ag_snd_ref.at[:, pl.ds(h * HN, HN)],
                dst_ref=ag_rcv_ref.at[r - 1, :, pl.ds(h * HN, HN)],
                send_sem=ag_send_sems.at[r - 1, h],
                recv_sem=ag_recv_sems.at[r - 1, h],
                device_id=(t,),
                device_id_type=pl.DeviceIdType.MESH,
            )

        for r in (2, 1, 3):
            t = (my + r) % N_DEV
            rs_snd_ref[r - 1, :, :] = jnp.dot(
                a_ref[pl.ds(t * CH, CH), :], b_ref[:, :],
                preferred_element_type=jnp.float32,
            ).astype(jnp.bfloat16)
            rs_rdma(r, 0).start()
        for r in (2, 1, 3):
            rs_rdma(r, 1).start()

        z = jnp.dot(
            a_ref[pl.ds(my * CH, CH), :], b_ref[:, :],
            preferred_element_type=jnp.float32,
        )

        for h in (0, 1):
            zh = z[:, h * HN:(h + 1) * HN]
            for q in (0, 2, 1):
                rs_rdma(q + 1, h).wait()
                zh = zh + rs_rcv_ref[q, :, h * HN:(h + 1) * HN].astype(jnp.float32)
            gh = _gelu(zh)
            ag_snd_ref[:, h * HN:(h + 1) * HN] = gh.astype(jnp.bfloat16)
            out_ref[my, :, h * HN:(h + 1) * HN] = gh
            for r in (2, 1, 3):
                ag_rdma(r, h).start()

        for h in (0, 1):
            for q in (0, 2, 1):
                ag_rdma(q + 1, h).wait()
                cq = (my - q - 1) % N_DEV
                out_ref[cq, :, h * HN:(h + 1) * HN] = (
                    ag_rcv_ref[q, :, h * HN:(h + 1) * HN].astype(jnp.float32)
                )

    out = pl.pallas_call(
        body,
        out_shape=jax.ShapeDtypeStruct((N_DEV, CH, N), jnp.float32),
        in_specs=[
            pl.BlockSpec(memory_space=pltpu.VMEM),
            pl.BlockSpec(memory_space=pltpu.VMEM),
        ],
        out_specs=pl.BlockSpec(memory_space=pltpu.VMEM),
        scratch_shapes=[
            pltpu.VMEM((N_DEV - 1, CH, N), jnp.bfloat16),
            pltpu.VMEM((N_DEV - 1, CH, N), jnp.bfloat16),
            pltpu.VMEM((CH, N), jnp.bfloat16),
            pltpu.VMEM((N_DEV - 1, CH, N), jnp.bfloat16),
            pltpu.SemaphoreType.DMA((N_DEV - 1, 2)),
            pltpu.SemaphoreType.DMA((N_DEV - 1, 2)),
            pltpu.SemaphoreType.DMA((N_DEV - 1, 2)),
            pltpu.SemaphoreType.DMA((N_DEV - 1, 2)),
        ],
        compiler_params=pltpu.CompilerParams(collective_id=0),
    )(A, B)
    return out.reshape(M, N)


# device time: 32265 ns/iter; 2.7085x vs baseline; 1.0304x over previous
import jax
import jax.numpy as jnp
from jax import lax
from jax.experimental import pallas as pl
from jax.experimental.pallas import tpu as pltpu

N_DEV = 4
M = 1024
N = 1024
CH = M // N_DEV
HN = N // 2

P1A, P1B, P2A, P2B, P3A, P3B, P4A, P4B = range(8)


def _gelu(z):
    return 0.5 * z * (1.0 + jnp.tanh(0.7978845608 * (z + 0.044715 * z * z * z)))


def kernel(A, B):
    def body(a_ref, b_ref, out_ref,
             p1a_snd, p1a_rcv, p1b_snd, p1b_rcv,
             acc_a, acc_b,
             p2a_snd, p2a_rcv, p2b_snd, p2b_rcv,
             ag_a, ag_b, p4a_rcv, p4b_rcv,
             send_sems, recv_sems):
        p = lax.axis_index("i")
        px = 3 - p
        py = p ^ 1
        cd = 3 - (p ^ 1)

        barrier_sem = pltpu.get_barrier_semaphore()
        for nbr in (px, py):
            pl.semaphore_signal(
                barrier_sem, inc=1,
                device_id=(nbr,), device_id_type=pl.DeviceIdType.MESH,
            )
        pl.semaphore_wait(barrier_sem, 2)

        def exch(idx, src, dst, target):
            return pltpu.make_async_remote_copy(
                src_ref=src, dst_ref=dst,
                send_sem=send_sems.at[idx], recv_sem=recv_sems.at[idx],
                device_id=(target,), device_id_type=pl.DeviceIdType.MESH,
            )

        def mm(c):
            return jnp.dot(
                a_ref[pl.ds(c * CH, CH), :], b_ref[:, :],
                preferred_element_type=jnp.float32,
            )

        zcd = mm(cd)
        p1a_snd[1, :, :] = zcd[:, :HN].astype(jnp.bfloat16)
        p1b_snd[1, :, :] = zcd[:, HN:].astype(jnp.bfloat16)
        zcx = mm(px)
        p1a_snd[0, :, :] = zcx[:, :HN].astype(jnp.bfloat16)
        rdma_p1a = exch(P1A, p1a_snd, p1a_rcv, px)
        rdma_p1a.start()
        acc_b[1, :, :] = zcx[:, HN:]
        zcy = mm(py)
        p1b_snd[0, :, :] = zcy[:, HN:].astype(jnp.bfloat16)
        rdma_p1b = exch(P1B, p1b_snd, p1b_rcv, py)
        rdma_p1b.start()
        acc_a[1, :, :] = zcy[:, :HN]
        zp = mm(p)
        acc_a[0, :, :] = zp[:, :HN]
        acc_b[0, :, :] = zp[:, HN:]

        rdma_p1a.wait()
        p2a_snd[:, :] = (
            acc_a[1, :, :] + p1a_rcv[1, :, :].astype(jnp.float32)
        ).astype(jnp.bfloat16)
        rdma_p2a = exch(P2A, p2a_snd, p2a_rcv, py)
        rdma_p2a.start()
        acc_a[0, :, :] = acc_a[0, :, :] + p1a_rcv[0, :, :].astype(jnp.float32)

        rdma_p1b.wait()
        p2b_snd[:, :] = (
            acc_b[1, :, :] + p1b_rcv[1, :, :].astype(jnp.float32)
        ).astype(jnp.bfloat16)
        rdma_p2b = exch(P2B, p2b_snd, p2b_rcv, px)
        rdma_p2b.start()
        acc_b[0, :, :] = acc_b[0, :, :] + p1b_rcv[0, :, :].astype(jnp.float32)

        rdma_p2a.wait()
        ga = _gelu(acc_a[0, :, :] + p2a_rcv[:, :].astype(jnp.float32))
        out_ref[p, :, :HN] = ga
        ag_a[0, :, :] = ga.astype(jnp.bfloat16)
        rdma_p3a = exch(P3A, ag_a.at[0], ag_a.at[1], py)
        rdma_p3a.start()

        rdma_p2b.wait()
        gb = _gelu(acc_b[0, :, :] + p2b_rcv[:, :].astype(jnp.float32))
        out_ref[p, :, HN:] = gb
        ag_b[0, :, :] = gb.astype(jnp.bfloat16)
        rdma_p3b = exch(P3B, ag_b.at[0], ag_b.at[1], px)
        rdma_p3b.start()

        rdma_p3a.wait()
        out_ref[py, :, :HN] = ag_a[1, :, :].astype(jnp.float32)
        rdma_p4a = exch(P4A, ag_a, p4a_rcv, px)
        rdma_p4a.start()

        rdma_p3b.wait()
        out_ref[px, :, HN:] = ag_b[1, :, :].astype(jnp.float32)
        rdma_p4b = exch(P4B, ag_b, p4b_rcv, py)
        rdma_p4b.start()

        rdma_p4a.wait()
        out_ref[px, :, :HN] = p4a_rcv[0, :, :].astype(jnp.float32)
        out_ref[cd, :, :HN] = p4a_rcv[1, :, :].astype(jnp.float32)

        rdma_p4b.wait()
        out_ref[py, :, HN:] = p4b_rcv[0, :, :].astype(jnp.float32)
        out_ref[cd, :, HN:] = p4b_rcv[1, :, :].astype(jnp.float32)

    bf = jnp.bfloat16
    out = pl.pallas_call(
        body,
        out_shape=jax.ShapeDtypeStruct((N_DEV, CH, N), jnp.float32),
        in_specs=[
            pl.BlockSpec(memory_space=pltpu.VMEM),
            pl.BlockSpec(memory_space=pltpu.VMEM),
        ],
        out_specs=pl.BlockSpec(memory_space=pltpu.VMEM),
        scratch_shapes=[
            pltpu.VMEM((2, CH, HN), bf),
            pltpu.VMEM((2, CH, HN), bf),
            pltpu.VMEM((2, CH, HN), bf),
            pltpu.VMEM((2, CH, HN), bf),
            pltpu.VMEM((2, CH, HN), jnp.float32),
            pltpu.VMEM((2, CH, HN), jnp.float32),
            pltpu.VMEM((CH, HN), bf),
            pltpu.VMEM((CH, HN), bf),
            pltpu.VMEM((CH, HN), bf),
            pltpu.VMEM((CH, HN), bf),
            pltpu.VMEM((2, CH, HN), bf),
            pltpu.VMEM((2, CH, HN), bf),
            pltpu.VMEM((2, CH, HN), bf),
            pltpu.VMEM((2, CH, HN), bf),
            pltpu.SemaphoreType.DMA((8,)),
            pltpu.SemaphoreType.DMA((8,)),
        ],
        compiler_params=pltpu.CompilerParams(collective_id=0),
    )(A, B)
    return out.reshape(M, N)
